# baseline (speedup 1.0000x reference)
import functools

import jax
import jax.numpy as jnp
from jax import lax
from jax.experimental import pallas as pl
from jax.experimental.pallas import tpu as pltpu
from jax.experimental.pallas import tpu_sc as plsc

VOCAB_SIZE = 1000000
EMBED_DIM = 64
BATCH = 16384

_info = plsc.get_sparse_core_info()
_NC = _info.num_cores
_NS = _info.num_subcores
_NW = _NC * _NS
_NBLOCKS = (VOCAB_SIZE + 127) // 128
_C_PER_W = 245
_CAP = 8
_NBUF = 5
_SWEEP_BLKS = _C_PER_W // _NBUF
_NGRP = BATCH // 16
_SENTINEL = jnp.int32(0x7FFFFFF)

_mesh = plsc.VectorSubcoreMesh(core_axis_name="c", subcore_axis_name="s")


@functools.partial(
    pl.kernel,
    mesh=_mesh,
    out_type=jax.ShapeDtypeStruct((BATCH * EMBED_DIM,), jnp.float32),
    scratch_types=[
        pltpu.VMEM((BATCH,), jnp.int32),
        pltpu.VMEM((_C_PER_W * _CAP * 16,), jnp.int32),
        pltpu.VMEM((_C_PER_W * 16,), jnp.int32),
        pltpu.VMEM((BATCH,), jnp.int32),
        pltpu.VMEM((BATCH,), jnp.int32),
        pltpu.VMEM((16,), jnp.int32),
        pltpu.VMEM((_NBUF, EMBED_DIM, 128), jnp.float32),
        pltpu.VMEM((32 * EMBED_DIM,), jnp.float32),
        pltpu.SemaphoreType.DMA,
    ]
    + [pltpu.SemaphoreType.DMA] * _NBUF,
    compiler_params=pltpu.CompilerParams(needs_layout_passes=False),
)
def _sweep_gather(
    idx_hbm, table_t_hbm, out_hbm,
    idx_v, bkt_v, cnt_v, lc_v, lp_v, tmp_v, ring_v, colw_v, wsem, *sems
):
    wid = lax.axis_index("s") * _NC + lax.axis_index("c")
    lo = wid * _C_PER_W
    lanes = lax.iota(jnp.int32, 16)

    pltpu.sync_copy(idx_hbm, idx_v)

    def zero_body(g, _):
        cnt_v[pl.ds(pl.multiple_of(g * 16, 16), 16)] = jnp.zeros(16, jnp.int32)
        return 0
    lax.fori_loop(0, _C_PER_W, zero_body, 0)

    def scan_body(g, n_left):
        v = idx_v[pl.ds(g * 16, 16)]
        blk = v >> 7
        col = v & 127
        mine = (blk >= lo) & (blk < lo + _C_PER_W)

        def with_matches(n_left):
            b = g * 16 + lanes
            packed = (b << 7) | col
            key = jnp.where(mine, blk, _SENTINEL)
            sk, sp = plsc.sort_key_val(key, packed)
            smine = sk != _SENTINEL
            tmp_v[...] = sk
            prev = plsc.load_gather(tmp_v, [jnp.maximum(lanes - 1, 0)])
            nxt = plsc.load_gather(tmp_v, [jnp.minimum(lanes + 1, 15)])
            boundary = (lanes == 0) | (sk != prev)
            seg0 = plsc.cummax(jnp.where(boundary, lanes, 0))
            rank = lanes - seg0
            is_last = ((lanes == 15) | (sk != nxt)) & smine
            rel = jnp.where(smine, sk - lo, 0)
            cnt = plsc.load_gather(cnt_v, [rel * 16])
            slot = cnt + rank
            ok = smine & (slot < _CAP)
            plsc.store_scatter(
                bkt_v, [(rel * _CAP + jnp.where(ok, slot, 0)) * 16], sp, mask=ok
            )
            plsc.addupdate_scatter(cnt_v, [rel * 16], rank + 1, mask=is_last)
            over = smine & (slot >= _CAP)
            over_i = over.astype(jnp.int32)
            lpos = n_left + plsc.cumsum(over_i) - over_i
            plsc.store_scatter(lc_v, [jnp.where(over, lpos, 0)], sk, mask=over)
            plsc.store_scatter(lp_v, [jnp.where(over, lpos, 0)], sp, mask=over)
            n_over = plsc.all_reduce_population_count(over)
            return n_left + n_over[0]

        return lax.cond(jnp.any(mine), with_matches, lambda n: n, n_left)

    n_left = lax.fori_loop(0, _NGRP, scan_body, jnp.int32(0))

    def fetch(cc, j):
        blk = jnp.minimum(lo + cc, _NBLOCKS - 1)
        off = pl.multiple_of(blk * 128, 128)
        pltpu.async_copy(table_t_hbm.at[:, pl.ds(off, 128)], ring_v.at[j], sems[j])

    def wait_slot(j):
        pltpu.make_async_copy(
            table_t_hbm.at[:, pl.ds(0, 128)], ring_v.at[j], sems[j]
        ).wait()

    def emit_column(slot_j, c, b, wcnt):
        ws = wcnt & 31
        colv = lax.broadcast(c, (16,))
        for s in range(EMBED_DIM // 16):
            vals = plsc.load_gather(ring_v.at[slot_j], [lanes + 16 * s, colv])
            wbase = ws * EMBED_DIM + 16 * s
            plsc.store_scatter(colw_v, [wbase + lanes], vals)
        @pl.when(wcnt >= 32)
        def _():
            pltpu.make_async_copy(
                colw_v.at[pl.ds(0, EMBED_DIM)],
                out_hbm.at[pl.ds(0, EMBED_DIM)],
                wsem,
            ).wait()
        pltpu.async_copy(
            colw_v.at[pl.ds(ws * EMBED_DIM, EMBED_DIM)],
            out_hbm.at[pl.ds(b * EMBED_DIM, EMBED_DIM)],
            wsem,
        )

    def process_block(cc, j, wcnt):
        rel16 = pl.multiple_of(cc * 16, 16)
        cv = cnt_v[pl.ds(rel16, 16)]
        n8 = jnp.minimum(cv[0], _CAP)
        for s in range(_CAP):
            @pl.when(s < n8)
            def _():
                ev = bkt_v[pl.ds(pl.multiple_of((cc * _CAP + s) * 16, 16), 16)]
                e = ev[0]
                emit_column(j, e & 127, e >> 7, wcnt + s)
        return wcnt + n8

    for j in range(_NBUF):
        fetch(jnp.int32(j), j)

    def sweep_body(blk_i, wcnt):
        for j in range(_NBUF):
            cc = blk_i * _NBUF + j
            wait_slot(j)
            wcnt = process_block(cc, j, wcnt)
            fetch(cc + _NBUF, j)
        return wcnt

    wcnt = lax.fori_loop(0, _SWEEP_BLKS - 1, sweep_body, jnp.int32(0))
    for j in range(_NBUF):
        cc = (_SWEEP_BLKS - 1) * _NBUF + j
        wait_slot(j)
        wcnt = process_block(jnp.int32(cc), j, wcnt)

    def left_body(k, wcnt):
        kal = pl.multiple_of((k >> 4) << 4, 16)
        lane = lax.broadcast(k & 15, (16,))
        tmp_v[...] = lc_v[pl.ds(kal, 16)]
        blk = plsc.load_gather(tmp_v, [lane])[0]
        tmp_v[...] = lp_v[pl.ds(kal, 16)]
        e = plsc.load_gather(tmp_v, [lane])[0]
        off = pl.multiple_of(blk * 128, 128)
        pltpu.sync_copy(table_t_hbm.at[:, pl.ds(off, 128)], ring_v.at[0])
        emit_column(0, e & 127, e >> 7, wcnt)
        return wcnt + 1

    wcnt = lax.fori_loop(0, n_left, left_body, wcnt)

    def drain_body(_, __):
        pltpu.make_async_copy(
            colw_v.at[pl.ds(0, EMBED_DIM)], out_hbm.at[pl.ds(0, EMBED_DIM)], wsem
        ).wait()
        return 0

    lax.fori_loop(0, jnp.minimum(wcnt, 32), drain_body, 0)


_B_PER_W = BATCH // _NW


@functools.partial(
    pl.kernel,
    mesh=_mesh,
    out_type=jax.ShapeDtypeStruct((EMBED_DIM, BATCH), jnp.float32),
    scratch_types=[
        pltpu.VMEM((_B_PER_W * EMBED_DIM,), jnp.float32),
        pltpu.VMEM((EMBED_DIM, _B_PER_W), jnp.float32),
    ],
    compiler_params=pltpu.CompilerParams(needs_layout_passes=False),
)
def _transpose(flat_hbm, out_t_hbm, buf_v, t_v):
    wid = lax.axis_index("s") * _NC + lax.axis_index("c")
    base = wid * _B_PER_W
    lanes = lax.iota(jnp.int32, 16)
    pltpu.sync_copy(
        flat_hbm.at[pl.ds(base * EMBED_DIM, _B_PER_W * EMBED_DIM)], buf_v
    )

    def body(k, _):
        kcol = lax.broadcast(k, (16,))
        for s in range(EMBED_DIM // 16):
            vals = buf_v[pl.ds(pl.multiple_of(k * EMBED_DIM + 16 * s, 16), 16)]
            plsc.store_scatter(t_v, [lanes + 16 * s, kcol], vals)
        return 0

    lax.fori_loop(0, _B_PER_W, body, 0)
    pltpu.sync_copy(t_v, out_t_hbm.at[:, pl.ds(base, _B_PER_W)])


def kernel(word_indices, word_embs):
    flat = _sweep_gather(word_indices.astype(jnp.int32), word_embs.T)
    return _transpose(flat).T


# speedup vs baseline: 1.2582x; 1.2582x over previous
import functools

import jax
import jax.numpy as jnp
from jax import lax
from jax.experimental import pallas as pl
from jax.experimental.pallas import tpu as pltpu
from jax.experimental.pallas import tpu_sc as plsc

VOCAB_SIZE = 1000000
EMBED_DIM = 64
BATCH = 16384

_info = plsc.get_sparse_core_info()
_NC = _info.num_cores
_NS = _info.num_subcores
_NW = _NC * _NS
_B_PER_W = BATCH // _NW
_NBLOCKS = (VOCAB_SIZE + 127) // 128
_C_PER_W = 245
_CAP = 8
_NBUF = 5
_SWEEP_BLKS = _C_PER_W // _NBUF
_SENTINEL = jnp.int32(0x7FFFFFF)

_mesh = plsc.VectorSubcoreMesh(core_axis_name="c", subcore_axis_name="s")


@functools.partial(
    pl.kernel,
    mesh=_mesh,
    out_type=(
        jax.ShapeDtypeStruct((_NW * _NW * _B_PER_W,), jnp.int32),
        jax.ShapeDtypeStruct((_NW * _NW * _B_PER_W,), jnp.int32),
        jax.ShapeDtypeStruct((_NW * _NW * 16,), jnp.int32),
    ),
    scratch_types=[
        pltpu.VMEM((_B_PER_W,), jnp.int32),
        pltpu.VMEM((_NW * 16,), jnp.int32),
        pltpu.VMEM((_NW * _B_PER_W,), jnp.int32),
        pltpu.VMEM((_NW * _B_PER_W,), jnp.int32),
        pltpu.VMEM((16,), jnp.int32),
        pltpu.SemaphoreType.DMA,
    ],
    compiler_params=pltpu.CompilerParams(needs_layout_passes=False),
)
def _binner(
    idx_hbm, bins_i_hbm, bins_b_hbm, cnt_hbm,
    idx_v, cnt_v, bi_v, bb_v, tmp_v, sem
):
    wid = lax.axis_index("s") * _NC + lax.axis_index("c")
    base = wid * _B_PER_W
    lanes = lax.iota(jnp.int32, 16)
    pltpu.sync_copy(idx_hbm.at[pl.ds(base, _B_PER_W)], idx_v)

    for o in range(_NW):
        cnt_v[pl.ds(o * 16, 16)] = jnp.zeros(16, jnp.int32)

    for g in range(_B_PER_W // 16):
        v = idx_v[pl.ds(g * 16, 16)]
        owner = (v >> 7) // _C_PER_W
        so, sl = plsc.sort_key_val(owner, lanes)
        tmp_v[...] = so
        prev = plsc.load_gather(tmp_v, [jnp.maximum(lanes - 1, 0)])
        nxt = plsc.load_gather(tmp_v, [jnp.minimum(lanes + 1, 15)])
        boundary = (lanes == 0) | (so != prev)
        seg0 = plsc.cummax(jnp.where(boundary, lanes, 0))
        rank = lanes - seg0
        is_last = (lanes == 15) | (so != nxt)
        cnt = plsc.load_gather(cnt_v, [so * 16])
        slot = cnt + rank
        tmp_v[...] = v
        iv = plsc.load_gather(tmp_v, [sl])
        bv = base + g * 16 + sl
        pos = so * _B_PER_W + slot
        plsc.store_scatter(bi_v, [pos], iv)
        plsc.store_scatter(bb_v, [pos], bv)
        plsc.addupdate_scatter(cnt_v, [so * 16], rank + 1, mask=is_last)

    pltpu.sync_copy(bi_v, bins_i_hbm.at[pl.ds(wid * _NW * _B_PER_W, _NW * _B_PER_W)])
    pltpu.sync_copy(bb_v, bins_b_hbm.at[pl.ds(wid * _NW * _B_PER_W, _NW * _B_PER_W)])
    for o in range(_NW):
        pltpu.async_copy(
            cnt_v.at[pl.ds(o * 16, 16)],
            cnt_hbm.at[pl.ds(o * _NW * 16 + wid * 16, 16)],
            sem,
        )
    for o in range(_NW):
        pltpu.make_async_copy(
            cnt_v.at[pl.ds(0, 16)], cnt_hbm.at[pl.ds(0, 16)], sem
        ).wait()


@functools.partial(
    pl.kernel,
    mesh=_mesh,
    out_type=jax.ShapeDtypeStruct((BATCH * EMBED_DIM,), jnp.float32),
    scratch_types=[
        pltpu.VMEM((_NW * 16,), jnp.int32),
        pltpu.VMEM((4, _B_PER_W,), jnp.int32),
        pltpu.VMEM((4, _B_PER_W,), jnp.int32),
        pltpu.VMEM((_C_PER_W * _CAP * 16,), jnp.int32),
        pltpu.VMEM((_C_PER_W * 16,), jnp.int32),
        pltpu.VMEM((BATCH,), jnp.int32),
        pltpu.VMEM((BATCH,), jnp.int32),
        pltpu.VMEM((16,), jnp.int32),
        pltpu.VMEM((_NBUF, EMBED_DIM, 128), jnp.float32),
        pltpu.VMEM((32 * EMBED_DIM,), jnp.float32),
        pltpu.SemaphoreType.DMA,
        pltpu.SemaphoreType.DMA,
    ]
    + [pltpu.SemaphoreType.DMA] * _NBUF,
    compiler_params=pltpu.CompilerParams(needs_layout_passes=False),
)
def _sweep_gather(
    table_t_hbm, bins_i_hbm, bins_b_hbm, cnt_hbm, out_hbm,
    myc_v, bi_v, bb_v, bkt_v, cnt_v, lc_v, lp_v, tmp_v, ring_v, colw_v,
    wsem, bsem, *sems
):
    wid = lax.axis_index("s") * _NC + lax.axis_index("c")
    lo = wid * _C_PER_W
    lanes = lax.iota(jnp.int32, 16)

    pltpu.sync_copy(cnt_hbm.at[pl.ds(wid * _NW * 16, _NW * 16)], myc_v)

    def zero_body(g, _):
        cnt_v[pl.ds(pl.multiple_of(g * 16, 16), 16)] = jnp.zeros(16, jnp.int32)
        return 0
    lax.fori_loop(0, _C_PER_W, zero_body, 0)

    def bin_fetch(src, j):
        off = src * _NW * _B_PER_W + wid * _B_PER_W
        pltpu.async_copy(bins_i_hbm.at[pl.ds(off, _B_PER_W)], bi_v.at[j], bsem)
        pltpu.async_copy(bins_b_hbm.at[pl.ds(off, _B_PER_W)], bb_v.at[j], bsem)

    def bin_wait(j):
        pltpu.make_async_copy(
            bins_i_hbm.at[pl.ds(0, _B_PER_W)], bi_v.at[j], bsem
        ).wait()
        pltpu.make_async_copy(
            bins_b_hbm.at[pl.ds(0, _B_PER_W)], bb_v.at[j], bsem
        ).wait()

    def merge_group(args):
        g, j, n_src, n_left = args
        vi = bi_v[j, pl.ds(g * 16, 16)]
        vb = bb_v[j, pl.ds(g * 16, 16)]
        valid = lanes < (n_src - g * 16)
        blk = vi >> 7
        col = vi & 127
        packed = (vb << 7) | col
        key = jnp.where(valid, blk, _SENTINEL)
        sk, sp = plsc.sort_key_val(key, packed)
        smine = sk != _SENTINEL
        tmp_v[...] = sk
        prev = plsc.load_gather(tmp_v, [jnp.maximum(lanes - 1, 0)])
        nxt = plsc.load_gather(tmp_v, [jnp.minimum(lanes + 1, 15)])
        boundary = (lanes == 0) | (sk != prev)
        seg0 = plsc.cummax(jnp.where(boundary, lanes, 0))
        rank = lanes - seg0
        is_last = ((lanes == 15) | (sk != nxt)) & smine
        rel = jnp.where(smine, sk - lo, 0)
        cnt = plsc.load_gather(cnt_v, [rel * 16])
        slot = cnt + rank
        ok = smine & (slot < _CAP)
        plsc.store_scatter(
            bkt_v, [(rel * _CAP + jnp.where(ok, slot, 0)) * 16], sp, mask=ok
        )
        plsc.addupdate_scatter(cnt_v, [rel * 16], rank + 1, mask=is_last)
        over = smine & (slot >= _CAP)
        over_i = over.astype(jnp.int32)
        lpos = n_left + plsc.cumsum(over_i) - over_i
        plsc.store_scatter(lc_v, [jnp.where(over, lpos, 0)], sk, mask=over)
        plsc.store_scatter(lp_v, [jnp.where(over, lpos, 0)], sp, mask=over)
        n_over = plsc.all_reduce_population_count(over)
        return n_left + n_over[0]

    for j in range(4):
        bin_fetch(jnp.int32(j), j)

    n_left = jnp.int32(0)
    for src in range(_NW):
        j = src % 4
        bin_wait(j)
        cv = myc_v[pl.ds(src * 16, 16)]
        n_src = cv[0]

        def mg_body(g, n_left, _j=j, _ns=None):
            return merge_group((g, _j, n_src, n_left))

        n_left = lax.fori_loop(0, (n_src + 15) >> 4, mg_body, n_left)
        if src + 4 < _NW:
            bin_fetch(jnp.int32(src + 4), j)

    def fetch(cc, j):
        blk = jnp.minimum(lo + cc, _NBLOCKS - 1)
        off = pl.multiple_of(blk * 128, 128)
        pltpu.async_copy(table_t_hbm.at[:, pl.ds(off, 128)], ring_v.at[j], sems[j])

    def wait_slot(j):
        pltpu.make_async_copy(
            table_t_hbm.at[:, pl.ds(0, 128)], ring_v.at[j], sems[j]
        ).wait()

    def emit_column(slot_j, c, b, wcnt):
        ws = wcnt & 31
        colv = lax.broadcast(c, (16,))
        for s in range(EMBED_DIM // 16):
            vals = plsc.load_gather(ring_v.at[slot_j], [lanes + 16 * s, colv])
            plsc.store_scatter(colw_v, [ws * EMBED_DIM + 16 * s + lanes], vals)
        @pl.when(wcnt >= 32)
        def _():
            pltpu.make_async_copy(
                colw_v.at[pl.ds(0, EMBED_DIM)],
                out_hbm.at[pl.ds(0, EMBED_DIM)],
                wsem,
            ).wait()
        pltpu.async_copy(
            colw_v.at[pl.ds(ws * EMBED_DIM, EMBED_DIM)],
            out_hbm.at[pl.ds(b * EMBED_DIM, EMBED_DIM)],
            wsem,
        )

    def process_block(cc, j, wcnt):
        rel16 = pl.multiple_of(cc * 16, 16)
        cv = cnt_v[pl.ds(rel16, 16)]
        n8 = jnp.minimum(cv[0], _CAP)
        for s in range(_CAP):
            @pl.when(s < n8)
            def _():
                ev = bkt_v[pl.ds(pl.multiple_of((cc * _CAP + s) * 16, 16), 16)]
                e = ev[0]
                emit_column(j, e & 127, e >> 7, wcnt + s)
        return wcnt + n8

    for j in range(_NBUF):
        fetch(jnp.int32(j), j)

    def sweep_body(blk_i, wcnt):
        for j in range(_NBUF):
            cc = blk_i * _NBUF + j
            wait_slot(j)
            wcnt = process_block(cc, j, wcnt)
            fetch(cc + _NBUF, j)
        return wcnt

    wcnt = lax.fori_loop(0, _SWEEP_BLKS - 1, sweep_body, jnp.int32(0))
    for j in range(_NBUF):
        cc = (_SWEEP_BLKS - 1) * _NBUF + j
        wait_slot(j)
        wcnt = process_block(jnp.int32(cc), j, wcnt)

    def left_body(k, wcnt):
        kal = pl.multiple_of((k >> 4) << 4, 16)
        lane = lax.broadcast(k & 15, (16,))
        tmp_v[...] = lc_v[pl.ds(kal, 16)]
        blk = plsc.load_gather(tmp_v, [lane])[0]
        tmp_v[...] = lp_v[pl.ds(kal, 16)]
        e = plsc.load_gather(tmp_v, [lane])[0]
        off = pl.multiple_of(blk * 128, 128)
        pltpu.sync_copy(table_t_hbm.at[:, pl.ds(off, 128)], ring_v.at[0])
        emit_column(0, e & 127, e >> 7, wcnt)
        return wcnt + 1

    wcnt = lax.fori_loop(0, n_left, left_body, wcnt)

    def drain_body(_, __):
        pltpu.make_async_copy(
            colw_v.at[pl.ds(0, EMBED_DIM)], out_hbm.at[pl.ds(0, EMBED_DIM)], wsem
        ).wait()
        return 0

    lax.fori_loop(0, jnp.minimum(wcnt, 32), drain_body, 0)


def kernel(word_indices, word_embs):
    idx32 = word_indices.astype(jnp.int32)
    bins_i, bins_b, counts = _binner(idx32)
    flat = _sweep_gather(word_embs.T, bins_i, bins_b, counts)
    return flat.reshape(BATCH, EMBED_DIM)
